# baseline (speedup 1.0000x reference)
import jax
import jax.numpy as jnp
from jax.experimental import pallas as pl
from jax.experimental.pallas import tpu as pltpu

_AFROMZERO = 0.02
_AFROMC = 0.01
_EXPMAX = 10.0
_OVER = 15.0


def _zmax(a, b, c):
    denom = c - a
    denom_safe = jnp.where(jnp.abs(denom) < 1e-9, 1e-9, denom)
    zmax_gen = 0.5 * (b + c - jnp.sqrt((b - c) ** 2 + 4.0 * a * b)) / denom_safe
    zmax_gen = jnp.where((zmax_gen > 0.9999) & (b > 100.0),
                         jnp.minimum(zmax_gen, 1.0 - a / jnp.maximum(b, 1e-12)),
                         zmax_gen)
    zmax_zero = jnp.where(c > b, b / jnp.maximum(c, 1e-12), 1.0)
    zmax_c = b / (b + c)
    return jnp.where(a < _AFROMZERO, zmax_zero,
                     jnp.where(jnp.abs(a - c) < _AFROMC, zmax_c, zmax_gen))


def _likelihood(z, mT, a, b_param, c=1.0):
    b = b_param * mT ** 2
    zmax = _zmax(a, b, c)
    aCoef = jnp.log(1.0 - z) - jnp.log(1.0 - zmax)
    bCoef = 1.0 / zmax - 1.0 / z
    cCoef = jnp.log(zmax) - jnp.log(z)
    fExp = b * bCoef + c * cCoef
    fExp = jnp.where(a < _AFROMZERO, fExp, fExp + a * aCoef)
    return jnp.exp(jnp.clip(fExp, -_EXPMAX, _EXPMAX))


def _prod_fold(w, axis):
    n = w.shape[axis]
    while n > 1:
        h = n // 2
        lo = jax.lax.slice_in_dim(w, 0, h, axis=axis)
        hi = jax.lax.slice_in_dim(w, h, n, axis=axis)
        w = lo * hi
        n = h
    return w


def _tc_body(z_ref, mT_ref, obs_ref, pv_ref, out_ref):
    z3 = z_ref[...]
    mT3 = mT_ref[...]
    obs = obs_ref[...]
    pa = pv_ref[0]
    pb = pv_ref[1]
    pa0 = pv_ref[2]
    pb0 = pv_ref[3]

    B, L, S = z3.shape
    valid3 = jax.lax.broadcasted_iota(jnp.int32, (B, L, 1), 1) < obs
    mask3 = valid3 & (z3 != 0.0)
    zs = jnp.where(mask3, z3, 0.5)
    mTs = jnp.where(valid3, mT3, 1.0)

    num = _likelihood(zs, mTs, pa, pb)
    den = _likelihood(zs, mTs, pa0, pb0)

    s_idx = jax.lax.broadcasted_iota(jnp.int32, (B, L, S), 2)
    w_acc = num / den
    w_rej = (_OVER - num) / (_OVER - den)
    w = jnp.where(mask3, jnp.where(s_idx == 0, w_acc, w_rej), 1.0)

    w = _prod_fold(w, 2)
    w = _prod_fold(w, 1)
    out_ref[...] = w


def kernel(z, mT, observable, params_a, params_b, params_base):
    B, L, S = z.shape
    pvec = jnp.stack([params_a, params_b, params_base[0], params_base[1]])
    obs3 = observable.reshape(B, 1, 1)
    mT3 = mT.reshape(B, L, 1)
    out = pl.pallas_call(
        _tc_body,
        out_shape=jax.ShapeDtypeStruct((B, 1, 1), jnp.float32),
        in_specs=[
            pl.BlockSpec(memory_space=pltpu.VMEM),
            pl.BlockSpec(memory_space=pltpu.VMEM),
            pl.BlockSpec(memory_space=pltpu.VMEM),
            pl.BlockSpec(memory_space=pltpu.SMEM),
        ],
        out_specs=pl.BlockSpec(memory_space=pltpu.VMEM),
    )(z, mT3, obs3, pvec)
    return out[:, 0, 0]


# speedup vs baseline: 1.0304x; 1.0304x over previous
import functools

import jax
import jax.numpy as jnp
from jax import lax
from jax.experimental import pallas as pl
from jax.experimental.pallas import tpu as pltpu
from jax.experimental.pallas import tpu_sc as plsc

_B, _L, _S = 16, 512, 16
_AFROMZERO = 0.02
_EXPMAX = 10.0
_OVER = 15.0
_LN2 = 0.6931471805599453

_LOGC = (
    2.643457467e-10, 0.9999999061, -0.5000000283, 0.3333473085,
    -0.250012529, 0.1994477744, -0.1657575011, 0.1505637611,
    -0.1429659855, 0.08383508477,
)


def _vlog(x):
    xi = lax.bitcast_convert_type(x, jnp.int32)
    e = (xi >> 23) - 127
    mi = (xi & 0x007FFFFF) | 0x3F800000
    m = lax.bitcast_convert_type(mi, jnp.float32)
    big = m > 1.4142135381698608
    m = jnp.where(big, m * 0.5, m)
    e = jnp.where(big, e + 1, e)
    t = m - 1.0
    p = jnp.full_like(x, _LOGC[9])
    for c in reversed(_LOGC[:9]):
        p = p * t + c
    return e.astype(jnp.float32) * _LN2 + p


def _vsqrt(x):
    xs = jnp.maximum(x, 1e-30)
    i = lax.bitcast_convert_type(xs, jnp.int32)
    y = lax.bitcast_convert_type(0x5F3759DF - (i >> 1), jnp.float32)
    y = y * (1.5 - 0.5 * xs * y * y)
    y = y * (1.5 - 0.5 * xs * y * y)
    y = y * (1.5 - 0.5 * xs * y * y)
    return xs * y


def _zmax_vec(a, b):
    denom = 1.0 - a
    denom_safe = jnp.where(jnp.abs(denom) < 1e-9, 1e-9, denom)
    bm1 = b - 1.0
    zmax_gen = 0.5 * (b + 1.0 - _vsqrt(bm1 * bm1 + 4.0 * a * b)) / denom_safe
    zmax_gen = jnp.where((zmax_gen > 0.9999) & (b > 100.0),
                         jnp.minimum(zmax_gen, 1.0 - a / jnp.maximum(b, 1e-12)),
                         zmax_gen)
    zmax_zero = jnp.where(b < 1.0, b, 1.0)
    zmax_c = b / (b + 1.0)
    return jnp.where(a < _AFROMZERO, zmax_zero,
                     jnp.where(jnp.abs(a - 1.0) < 0.01, zmax_c, zmax_gen))


def _sc_body(z_h, mT_h, obs_h, par_h, out_h, z_v, mT_v, obs_v, par_v, io_v,
             part_v):
    c = lax.axis_index("c")
    s = lax.axis_index("s")
    e = c * 8 + s // 2
    h = s % 2

    pltpu.sync_copy(z_h.at[pl.ds(e * (_L * _S) + h * (_L * _S // 2), _L * _S // 2)], z_v)
    pltpu.sync_copy(mT_h.at[pl.ds(e * _L + h * (_L // 2), _L // 2)], mT_v)
    pltpu.sync_copy(obs_h.at[pl.ds(e * 16, 16)], obs_v)
    pltpu.sync_copy(par_h, par_v)

    obs_vec = obs_v[...]
    a1 = par_v[pl.ds(0, 16)]
    bp1 = par_v[pl.ds(16, 16)]
    a0 = par_v[pl.ds(32, 16)]
    bp0 = par_v[pl.ds(48, 16)]
    ae1 = jnp.where(a1 < _AFROMZERO, 0.0, a1)
    ae0 = jnp.where(a0 < _AFROMZERO, 0.0, a0)

    lane = lax.iota(jnp.int32, 16)
    lane16 = lane * 16
    row0 = h * (_L // 2)

    def _prep(a, aeff, b):
        zm = _zmax_vec(a, b)
        linv = 1.0 / zm
        lzm = _vlog(zm)
        l1zm = _vlog(jnp.maximum(1.0 - zm, 1e-38))
        k = b * linv + lzm - aeff * l1zm
        return k

    def _block(i, lacc):
        mTv = mT_v[pl.ds(i * 16, 16)]
        valid = (lane + (row0 + i * 16)) < obs_vec
        mt2 = mTv * mTv
        b1 = bp1 * mt2
        b0 = bp0 * mt2
        k1 = _prep(a1, ae1, b1)
        k0 = _prep(a0, ae0, b0)

        bp = jnp.ones(16, jnp.float32)
        for scol in range(_S):
            zr = plsc.load_gather(z_v, [lane16 + (i * 256 + scol)])
            m = valid & (zr != 0.0)
            zs = jnp.where(m, zr, 0.5)
            lz = _vlog(zs)
            l1z = _vlog(1.0 - zs)
            invz = 1.0 / zs
            f1 = jnp.clip(k1 - b1 * invz - lz + ae1 * l1z, -_EXPMAX, _EXPMAX)
            f0 = jnp.clip(k0 - b0 * invz - lz + ae0 * l1z, -_EXPMAX, _EXPMAX)
            if scol == 0:
                w = jnp.exp(f1 - f0)
            else:
                w = (_OVER - jnp.exp(f1)) / (_OVER - jnp.exp(f0))
            bp = bp * jnp.where(m, w, 1.0)
        return lacc + _vlog(bp)

    lacc = lax.fori_loop(0, _L // 32, _block, jnp.zeros(16, jnp.float32))

    io_v[...] = lacc
    pltpu.sync_copy(io_v, out_h.at[16 + c * 16 + s])
    plsc.subcore_barrier()

    @pl.when(h == 0)
    def _():
        pltpu.sync_copy(out_h.at[16 + c * 16 + s + 1], part_v)
        tot = lacc + part_v[...]
        total = jnp.sum(tot)
        io_v[...] = jnp.exp(jnp.full((16,), total, jnp.float32))
        pltpu.sync_copy(io_v, out_h.at[e])


@functools.cache
def _make_sc_kernel():
    return pl.kernel(
        _sc_body,
        out_type=jax.ShapeDtypeStruct((_B + 32, 16), jnp.float32),
        mesh=plsc.VectorSubcoreMesh(core_axis_name="c", subcore_axis_name="s",
                                    num_cores=2, num_subcores=16),
        compiler_params=pltpu.CompilerParams(needs_layout_passes=False),
        scratch_types=[
            pltpu.VMEM((_L * _S // 2,), jnp.float32),
            pltpu.VMEM((_L // 2,), jnp.float32),
            pltpu.VMEM((16,), jnp.int32),
            pltpu.VMEM((64,), jnp.float32),
            pltpu.VMEM((16,), jnp.float32),
            pltpu.VMEM((16,), jnp.float32),
        ],
    )


def kernel(z, mT, observable, params_a, params_b, params_base):
    zf = z.reshape(-1)
    mTf = mT.reshape(-1)
    obs_b = jnp.broadcast_to(observable[:, None], (_B, 16)).reshape(-1)
    par = jnp.repeat(
        jnp.stack([params_a, params_b, params_base[0], params_base[1]]), 16)
    out = _make_sc_kernel()(zf, mTf, obs_b, par)
    return out[:_B, 0]
